# Initial kernel scaffold; baseline (speedup 1.0000x reference)
import jax
import jax.numpy as jnp
from jax.experimental import pallas as pl


def kernel(features, edge_index, W1, b1, W2, b2):
    raise NotImplementedError("write your pallas kernel here")


# speedup vs baseline: 10.2150x; 10.2150x over previous
import functools

import jax
import jax.numpy as jnp
from jax import lax
from jax.experimental import pallas as pl
from jax.experimental.pallas import tpu as pltpu
from jax.experimental.pallas import tpu_sc as plsc

N = 10000
E = 320000
D = 128
NP = 10240
NPH = NP // 2
NC = 2
NS = 16
NW = NC * NS
EP = 327680
K = 128
RD = 80
RA = 160
SEG = 40
QR = 8
ASH = NPH + 256
AZT = ASH // NS

_mesh = plsc.VectorSubcoreMesh(core_axis_name="c", subcore_axis_name="s")
_HIGH = lax.Precision.HIGHEST
_SC_PARAMS = pltpu.CompilerParams(needs_layout_passes=False)


def _zero_1d(ref, n):
    z = jnp.zeros((16,), ref.dtype)

    def body(i, _):
        ref[pl.ds(i * 16, 16)] = z
        return 0

    lax.fori_loop(0, n // 16, body, 0)


@functools.partial(
    pl.kernel,
    out_type=jax.ShapeDtypeStruct((2, NW, NP), jnp.float32),
    mesh=_mesh,
    compiler_params=_SC_PARAMS,
    scratch_types=[
        pltpu.VMEM((RD, K), jnp.int32),
        pltpu.VMEM((RD, K), jnp.int32),
        pltpu.VMEM((NP,), jnp.float32),
        pltpu.VMEM((NP,), jnp.float32),
    ],
)
def _sc_degrees(edges, out, srcv, dstv, dego, degi):
    cid = lax.axis_index("c")
    sid = lax.axis_index("s")
    wid = cid * NS + sid
    pltpu.sync_copy(edges.at[0, wid], srcv)
    pltpu.sync_copy(edges.at[1, wid], dstv)
    _zero_1d(dego, NP)
    _zero_1d(degi, NP)
    ones = jnp.ones((16,), jnp.float32)
    lanes = lax.iota(jnp.int32, 16)
    tile_base = wid * (RD * K)

    def body(m, _):
        for q in range(K // 16):
            pos = tile_base + m * K + q * 16 + lanes
            live = pos < E
            s16 = srcv[m, pl.ds(q * 16, 16)]
            d16 = dstv[m, pl.ds(q * 16, 16)]
            plsc.addupdate_scatter(dego, [s16], ones, mask=live)
            plsc.addupdate_scatter(degi, [d16], ones, mask=live)
        return 0

    lax.fori_loop(0, RD, body, 0)
    pltpu.sync_copy(dego, out.at[0, wid])
    pltpu.sync_copy(degi, out.at[1, wid])


def _isqrt_body(dp_ref, isq_ref):
    s = jnp.sum(dp_ref[...], axis=1)
    isq_ref[...] = lax.rsqrt(jnp.maximum(s, 1.0))


def _tc_isqrt(deg_p):
    return pl.pallas_call(
        _isqrt_body,
        out_shape=jax.ShapeDtypeStruct((2, NP), jnp.float32),
    )(deg_p)


def _scale_body(f_ref, oi_ref, out_ref):
    out_ref[...] = f_ref[...] * oi_ref[...]


def _tc_scale(features, oi_col):
    br = 2000
    return pl.pallas_call(
        _scale_body,
        grid=(N // br,),
        in_specs=[
            pl.BlockSpec((br, D), lambda i: (i, 0)),
            pl.BlockSpec((br, 1), lambda i: (i, 0)),
        ],
        out_specs=pl.BlockSpec((br, D), lambda i: (i, 0)),
        out_shape=jax.ShapeDtypeStruct((N, D), jnp.float32),
    )(features, oi_col)


@functools.partial(
    pl.kernel,
    out_type=[
        jax.ShapeDtypeStruct((NC, NPH, D), jnp.float32),
        jax.ShapeDtypeStruct((NW, NP), jnp.float32),
    ],
    mesh=_mesh,
    compiler_params=_SC_PARAMS,
    scratch_types=[
        pltpu.VMEM((SEG, K), jnp.int32),
        pltpu.VMEM((SEG, K), jnp.int32),
        pltpu.VMEM((QR, K), jnp.int32),
        pltpu.VMEM((QR, K), jnp.int32),
        pltpu.VMEM((NP,), jnp.float32),
        pltpu.VMEM((NP,), jnp.float32),
        pltpu.VMEM((K, D), jnp.float32),
        pltpu.VMEM_SHARED((ASH, D), jnp.float32),
        pltpu.SemaphoreType.DMA,
    ],
)
def _sc_aggregate(edges, h0n, isq, agg_out, c_out, srcv, dstv, qsrc, qdst,
                  iiv, cpart, rows, agg_sh, sem):
    cid = lax.axis_index("c")
    sid = lax.axis_index("s")
    wid = cid * NS + sid
    pltpu.sync_copy(isq.at[1], iiv)
    _zero_1d(cpart, NP)

    z = jnp.zeros((16,), jnp.float32)

    def zrow(r, _):
        for q in range(D // 16):
            rows[r, pl.ds(q * 16, 16)] = z
        return 0

    lax.fori_loop(0, K, zrow, 0)
    pltpu.sync_copy(rows, agg_sh.at[pl.ds(sid * AZT, K)])
    pltpu.sync_copy(rows, agg_sh.at[pl.ds(sid * AZT + K, K)])
    pltpu.sync_copy(rows.at[pl.ds(0, AZT - 2 * K)],
                    agg_sh.at[pl.ds(sid * AZT + 2 * K, AZT - 2 * K)])
    plsc.subcore_barrier()

    lanes = lax.iota(jnp.int32, 16)
    chunk_base = sid * (RA * K)
    dst_lo = cid * NPH

    def drain_one(dr):
        j = jnp.bitwise_and(dr, QR - 1)
        pltpu.async_copy(h0n.at[qsrc.at[j]], rows, sem).wait()
        pltpu.sync_copy(rows, agg_sh.at[qdst.at[j]], add=True)
        return dr + 1

    def seg_body(g, carry):
        pltpu.sync_copy(edges.at[0, sid, pl.ds(g * SEG, SEG)], srcv)
        pltpu.sync_copy(edges.at[1, sid, pl.ds(g * SEG, SEG)], dstv)
        seg_base = chunk_base + g * (SEG * K)

        def fbody(mm, carry2):
            cnt, drained = carry2
            for q in range(K // 16):
                pos = seg_base + mm * K + q * 16 + lanes
                live = pos < E
                s16 = srcv[mm, pl.ds(q * 16, 16)]
                d16 = dstv[mm, pl.ds(q * 16, 16)]
                upper = d16 >= NPH
                mine = jnp.logical_and(upper == (cid > 0), live)
                w = plsc.load_gather(iiv, [d16])
                plsc.addupdate_scatter(cpart, [s16], w, mask=mine)
                pc = plsc.cumsum(mine.astype(jnp.int32))
                slot = cnt + pc - 1
                row = jnp.bitwise_and(lax.shift_right_logical(slot, 7), QR - 1)
                col = jnp.bitwise_and(slot, K - 1)
                plsc.store_scatter(qsrc, [row, col], s16, mask=mine)
                plsc.store_scatter(qdst, [row, col], d16 - dst_lo, mask=mine)
                cnt = cnt + jnp.sum(mine.astype(jnp.int32))
            drained = lax.cond(lax.shift_right_logical(cnt, 7) > drained,
                               drain_one, lambda d: d, drained)
            return cnt, drained

        return lax.fori_loop(0, SEG, fbody, carry)

    cnt, drained = lax.fori_loop(
        0, RA // SEG, seg_body, (jnp.int32(0), jnp.int32(0)))

    cnt_up = jnp.bitwise_and(cnt + (K - 1), jnp.int32(~(K - 1)))
    for t in range(K // 16):
        slot = cnt + t * 16 + lanes
        pad = slot < cnt_up
        row = jnp.bitwise_and(lax.shift_right_logical(slot, 7), QR - 1)
        col = jnp.bitwise_and(slot, K - 1)
        plsc.store_scatter(qsrc, [row, col], jnp.zeros((16,), jnp.int32),
                           mask=pad)
        plsc.store_scatter(qdst, [row, col],
                           jnp.full((16,), NPH, jnp.int32), mask=pad)

    def fl(i, dr):
        return drain_one(dr)

    lax.fori_loop(0, lax.shift_right_logical(cnt_up, 7) - drained, fl,
                  drained)
    plsc.subcore_barrier()
    pltpu.sync_copy(cpart, c_out.at[wid])
    pltpu.sync_copy(agg_sh.at[pl.ds(sid * (NPH // NS), NPH // NS)],
                    agg_out.at[cid, pl.ds(sid * (NPH // NS), NPH // NS)])


def _finish_body(agg_ref, ii_ref, oi_ref, c_ref, w1_ref, b1_ref, w2_ref,
                 b2_ref, out_ref, s_ref):
    i = pl.program_id(0)

    @pl.when(i == 0)
    def _():
        s_ref[...] = jnp.zeros_like(s_ref)

    aggn = agg_ref[...] * ii_ref[...]
    h1 = jnp.maximum(
        jnp.dot(aggn, w1_ref[...], precision=_HIGH) + b1_ref[...], 0.0)
    h1n = h1 * oi_ref[...]
    cs = jnp.sum(c_ref[...], axis=0, keepdims=True)
    s_ref[...] += jnp.dot(cs, h1n, precision=_HIGH)

    @pl.when(i == pl.num_programs(0) - 1)
    def _():
        out_ref[...] = (
            jnp.dot(s_ref[...] * (1.0 / N), w2_ref[...], precision=_HIGH)
            + b2_ref[...])


def _tc_finish(agg, ii_col, oi_col, c_p, W1, b1r, W2, b2r):
    br = 512
    return pl.pallas_call(
        _finish_body,
        grid=(NP // br,),
        in_specs=[
            pl.BlockSpec((br, D), lambda i: (i, 0)),
            pl.BlockSpec((br, 1), lambda i: (i, 0)),
            pl.BlockSpec((br, 1), lambda i: (i, 0)),
            pl.BlockSpec((NW, br), lambda i: (0, i)),
            pl.BlockSpec((D, D), lambda i: (0, 0)),
            pl.BlockSpec((1, D), lambda i: (0, 0)),
            pl.BlockSpec((D, D), lambda i: (0, 0)),
            pl.BlockSpec((1, D), lambda i: (0, 0)),
        ],
        out_specs=pl.BlockSpec((1, D), lambda i: (0, 0)),
        out_shape=jax.ShapeDtypeStruct((1, D), jnp.float32),
        scratch_shapes=[pltpu.VMEM((1, D), jnp.float32)],
    )(agg, ii_col, oi_col, c_p, W1, b1r, W2, b2r)


@jax.jit
def kernel(features, edge_index, W1, b1, W2, b2):
    ep = jnp.concatenate(
        [edge_index.astype(jnp.int32),
         jnp.zeros((2, EP - E), jnp.int32)], axis=1)
    deg_p = _sc_degrees(ep.reshape(2, NW, RD, K))
    isq = _tc_isqrt(deg_p)
    oi_n = isq[0, :N].reshape(N, 1)
    h0n = _tc_scale(features, oi_n)
    agg_p, c_p = _sc_aggregate(ep.reshape(2, NS, RA, K), h0n, isq)
    agg = agg_p.reshape(NP, D)
    ii_col = isq[1].reshape(NP, 1)
    oi_col = isq[0].reshape(NP, 1)
    return _tc_finish(agg, ii_col, oi_col, c_p, W1,
                      b1.reshape(1, D), W2, b2.reshape(1, D))
